# baseline (speedup 1.0000x reference)
import functools

import jax
import jax.numpy as jnp
from jax import lax
from jax.experimental import pallas as pl
from jax.experimental.pallas import tpu as pltpu
from jax.experimental.pallas import tpu_sc as plsc

N = 10000
E = 160000
D = 128
H = 128
C = 40
NEG_SLOPE = 0.01
EPS = 1e-5

NS = 16
CHUNK = 128
N_ACC = 10112
ROWS_PT = N_ACC // NS
E_PAD = 163840
EC_PT = E_PAD // NS // CHUNK
IDX_ROWS = E_PAD // CHUNK
DEG_ROWS_PT = 2 * E_PAD // NS // CHUNK
DEG_ACC = 20480
DEG_PT = DEG_ACC // NS

_PREC = jax.lax.Precision.HIGHEST
_mesh = plsc.VectorSubcoreMesh(core_axis_name="c", subcore_axis_name="s")


def _mm(a, b):
    return jnp.dot(a, b, precision=_PREC, preferred_element_type=jnp.float32)


def _bn(h, g, b):
    mu = jnp.mean(h, axis=0, keepdims=True)
    var = jnp.mean((h - mu) ** 2, axis=0, keepdims=True)
    return g * (h - mu) * lax.rsqrt(var + EPS) + b


@functools.partial(
    pl.kernel,
    out_type=jax.ShapeDtypeStruct((2, DEG_ACC), jnp.float32),
    mesh=_mesh,
    scratch_types=[
        pltpu.VMEM((DEG_ROWS_PT, CHUNK), jnp.int32),
        pltpu.VMEM((CHUNK,), jnp.float32),
        pltpu.VMEM_SHARED((DEG_ACC,), jnp.float32),
    ],
)
def _sc_degrees(idx_hbm, zeros_hbm, ones_hbm, out_hbm, idx_v, ones_v, acc_sh):
    c = lax.axis_index("c")
    s = lax.axis_index("s")
    pltpu.sync_copy(idx_hbm.at[pl.ds(c * 2 * IDX_ROWS + s * DEG_ROWS_PT, DEG_ROWS_PT)], idx_v)
    pltpu.sync_copy(ones_hbm, ones_v)
    pltpu.sync_copy(zeros_hbm, acc_sh.at[pl.ds(s * DEG_PT, DEG_PT)])
    plsc.subcore_barrier()

    @pl.loop(0, DEG_ROWS_PT)
    def _(j):
        pltpu.sync_copy(ones_v, acc_sh.at[idx_v.at[j]], add=True)

    plsc.subcore_barrier()
    pltpu.sync_copy(acc_sh.at[pl.ds(s * DEG_PT, DEG_PT)],
                    out_hbm.at[c].at[pl.ds(s * DEG_PT, DEG_PT)])


@functools.partial(
    pl.kernel,
    out_type=jax.ShapeDtypeStruct((2 * N_ACC, H), jnp.float32),
    mesh=_mesh,
    scratch_types=[
        pltpu.VMEM((EC_PT, CHUNK), jnp.int32),
        pltpu.VMEM((8, CHUNK), jnp.int32),
        pltpu.VMEM((CHUNK, H), jnp.float32),
        pltpu.VMEM((CHUNK, H), jnp.float32),
        pltpu.VMEM_SHARED((N_ACC, H), jnp.float32),
        pltpu.SemaphoreType.DMA,
        pltpu.SemaphoreType.DMA,
        pltpu.SemaphoreType.DMA,
        pltpu.SemaphoreType.DMA,
    ],
)
def _sc_scatter(y_hbm, src_hbm, dst_hbm, zeros_hbm, out_hbm,
                src_v, dst_v, rows0_v, rows1_v, acc_sh,
                semg0, semg1, sems0, sems1):
    c = lax.axis_index("c")
    s = lax.axis_index("s")
    row0 = c * IDX_ROWS + s * EC_PT
    pltpu.sync_copy(src_hbm.at[pl.ds(row0, EC_PT)], src_v)
    pltpu.sync_copy(zeros_hbm.at[pl.ds(0, ROWS_PT)],
                    acc_sh.at[pl.ds(s * ROWS_PT, ROWS_PT)])
    plsc.subcore_barrier()

    pltpu.async_copy(y_hbm.at[src_v.at[0]], rows0_v, semg0)

    @pl.loop(0, EC_PT, step=8)
    def _(j):
        @pl.when(j > 0)
        def _():
            pltpu.make_async_copy(rows1_v, acc_sh.at[dst_v.at[7]], sems1).wait()

        pltpu.sync_copy(dst_hbm.at[pl.ds(row0 + j, 8)], dst_v)
        for k in range(0, 8, 2):
            pltpu.make_async_copy(y_hbm.at[src_v.at[j + k]], rows0_v, semg0).wait()
            pltpu.async_copy(rows0_v, acc_sh.at[dst_v.at[k]], sems0, add=True)
            if k > 0:
                pltpu.make_async_copy(rows1_v, acc_sh.at[dst_v.at[k - 1]], sems1).wait()
            pltpu.async_copy(y_hbm.at[src_v.at[j + k + 1]], rows1_v, semg1)
            pltpu.make_async_copy(y_hbm.at[src_v.at[j + k + 1]], rows1_v, semg1).wait()
            pltpu.async_copy(rows1_v, acc_sh.at[dst_v.at[k + 1]], sems1, add=True)
            pltpu.make_async_copy(rows0_v, acc_sh.at[dst_v.at[k]], sems0).wait()

            @pl.when(j + k + 2 < EC_PT)
            def _():
                pltpu.async_copy(y_hbm.at[src_v.at[j + k + 2]], rows0_v, semg0)

    pltpu.make_async_copy(rows1_v, acc_sh.at[dst_v.at[7]], sems1).wait()
    plsc.subcore_barrier()
    pltpu.sync_copy(acc_sh.at[pl.ds(s * ROWS_PT, ROWS_PT)],
                    out_hbm.at[pl.ds(c * N_ACC + s * ROWS_PT, ROWS_PT)])


def _tc_pre(x_ref, w_ref, b_ref, g_ref, be_ref, o_ref):
    h = _mm(x_ref[...], w_ref[...]) + b_ref[...]
    o_ref[...] = jnp.maximum(_bn(h, g_ref[...], be_ref[...]), 0.0)


def _tc_a(h_ref, deg_ref, wsk_ref, bsk_ref, we0_ref, we1_ref, y_ref, hres_ref):
    h = h_ref[...]
    deg = deg_ref[...]
    hres_ref[...] = _mm(h, wsk_ref[...]) + bsk_ref[...]
    c0 = lax.rsqrt(jnp.clip(deg[0, 0:N], 1.0, None)).reshape(N, 1)
    c1 = lax.rsqrt(jnp.clip(deg[1, 0:N], 1.0, None)).reshape(N, 1)
    y_ref[0:N, :] = _mm(h * c0, we0_ref[...])
    y_ref[N:N_ACC, :] = jnp.zeros((N_ACC - N, H), jnp.float32)
    y_ref[N_ACC:N_ACC + N, :] = _mm(h * c1, we1_ref[...])
    y_ref[N_ACC + N:, :] = jnp.zeros((N_ACC - N, H), jnp.float32)


def _tc_b(s_ref, deg_ref, hres_ref, b0_ref, b1_ref, g_ref, be_ref, o_ref):
    deg = deg_ref[...]
    i0 = lax.rsqrt(jnp.clip(deg[0, N_ACC:N_ACC + N], 1.0, None)).reshape(N, 1)
    i1 = lax.rsqrt(jnp.clip(deg[1, N_ACC:N_ACC + N], 1.0, None)).reshape(N, 1)
    rel0 = s_ref[0:N, :] * i0 + b0_ref[...]
    rel1 = s_ref[N_ACC:N_ACC + N, :] * i1 + b1_ref[...]
    t = 0.5 * (rel0 + rel1) + hres_ref[...]
    t = _bn(t, g_ref[...], be_ref[...])
    o_ref[...] = jnp.where(t >= 0.0, t, NEG_SLOPE * t)


def _tc_head(h_ref, w1_ref, b1_ref, g_ref, be_ref, w2_ref, b2_ref, o_ref):
    t = _mm(h_ref[...], w1_ref[...]) + b1_ref[...]
    t = jnp.maximum(_bn(t, g_ref[...], be_ref[...]), 0.0)
    o_ref[...] = _mm(t, w2_ref[...]) + b2_ref[...]


def _call(body, out_shapes):
    return pl.pallas_call(body, out_shape=out_shapes)


def kernel(x, edge_index_e0, edge_index_e1, params):
    p = params
    f32 = jnp.float32
    pad = jnp.full((E_PAD - E,), N, jnp.int32)
    s0p = jnp.concatenate([edge_index_e0[0], pad])
    d0p = jnp.concatenate([edge_index_e0[1], pad])
    s1p = jnp.concatenate([edge_index_e1[0], pad])
    d1p = jnp.concatenate([edge_index_e1[1], pad])

    deg_idx = jnp.stack([
        jnp.concatenate([s0p, d0p + N_ACC]),
        jnp.concatenate([s1p, d1p + N_ACC]),
    ]).reshape(4 * IDX_ROWS, CHUNK)
    src_all = jnp.concatenate([s0p, s1p + N_ACC]).reshape(2 * IDX_ROWS, CHUNK)
    dst_all = jnp.concatenate([d0p, d1p]).reshape(2 * IDX_ROWS, CHUNK)

    zeros1d = jnp.zeros((DEG_PT,), f32)
    ones1d = jnp.ones((CHUNK,), f32)
    zeros2d = jnp.zeros((ROWS_PT, H), f32)

    deg = _sc_degrees(deg_idx, zeros1d, ones1d)

    def r2(v):
        return v.reshape(1, -1)

    h = _call(_tc_pre, jax.ShapeDtypeStruct((N, H), f32))(
        x, p['W_fr'], r2(p['b_fr']), r2(p['g_fr']), r2(p['be_fr']))

    for lp in p['layers']:
        y, hres = _call(_tc_a, [
            jax.ShapeDtypeStruct((2 * N_ACC, H), f32),
            jax.ShapeDtypeStruct((N, H), f32),
        ])(h, deg, lp['W_skip'], r2(lp['b_skip']), lp['W_e0'], lp['W_e1'])
        s_out = _sc_scatter(y, src_all, dst_all, zeros2d)
        h = _call(_tc_b, jax.ShapeDtypeStruct((N, H), f32))(
            s_out, deg, hres, r2(lp['b_e0']), r2(lp['b_e1']),
            r2(lp['g']), r2(lp['be']))

    out = _call(_tc_head, jax.ShapeDtypeStruct((N, C), f32))(
        h, p['W_c1'], r2(p['b_c1']), r2(p['g_c']), r2(p['be_c']),
        p['W_c2'], r2(p['b_c2']))
    return out


# speedup vs baseline: 2.0130x; 2.0130x over previous
import functools

import jax
import jax.numpy as jnp
from jax import lax
from jax.experimental import pallas as pl
from jax.experimental.pallas import tpu as pltpu
from jax.experimental.pallas import tpu_sc as plsc

N = 10000
E = 160000
D = 128
H = 128
C = 40
NEG_SLOPE = 0.01
EPS = 1e-5

NS = 16
CHUNK = 128
N_ACC = 10112
ROWS_PT = N_ACC // NS
E_PAD = 163840
EC_PT = E_PAD // NS // CHUNK
IDX_ROWS = E_PAD // CHUNK
DEG_ROWS_PT = 2 * E_PAD // NS // CHUNK
DEG_ACC = 20480
DEG_PT = DEG_ACC // NS

_PREC = jax.lax.Precision.HIGHEST
_mesh = plsc.VectorSubcoreMesh(core_axis_name="c", subcore_axis_name="s")


def _mm(a, b):
    return jnp.dot(a, b, precision=_PREC, preferred_element_type=jnp.float32)


def _bn(h, g, b):
    mu = jnp.mean(h, axis=0, keepdims=True)
    var = jnp.mean((h - mu) ** 2, axis=0, keepdims=True)
    return g * (h - mu) * lax.rsqrt(var + EPS) + b


@functools.partial(
    pl.kernel,
    out_type=jax.ShapeDtypeStruct((2, DEG_ACC), jnp.float32),
    mesh=_mesh,
    scratch_types=[
        pltpu.VMEM((DEG_ROWS_PT, CHUNK), jnp.int32),
        pltpu.VMEM((CHUNK,), jnp.float32),
        pltpu.VMEM_SHARED((DEG_ACC,), jnp.float32),
    ],
)
def _sc_degrees(idx_hbm, zeros_hbm, ones_hbm, out_hbm, idx_v, ones_v, acc_sh):
    c = lax.axis_index("c")
    s = lax.axis_index("s")
    pltpu.sync_copy(idx_hbm.at[pl.ds(c * 2 * IDX_ROWS + s * DEG_ROWS_PT, DEG_ROWS_PT)], idx_v)
    pltpu.sync_copy(ones_hbm, ones_v)
    pltpu.sync_copy(zeros_hbm, acc_sh.at[pl.ds(s * DEG_PT, DEG_PT)])
    plsc.subcore_barrier()

    @pl.loop(0, DEG_ROWS_PT)
    def _(j):
        pltpu.sync_copy(ones_v, acc_sh.at[idx_v.at[j]], add=True)

    plsc.subcore_barrier()
    pltpu.sync_copy(acc_sh.at[pl.ds(s * DEG_PT, DEG_PT)],
                    out_hbm.at[c].at[pl.ds(s * DEG_PT, DEG_PT)])


@functools.partial(
    pl.kernel,
    out_type=jax.ShapeDtypeStruct((2 * N_ACC, H), jnp.float32),
    mesh=_mesh,
    scratch_types=[
        pltpu.VMEM((EC_PT, CHUNK), jnp.int32),
        pltpu.VMEM((8, CHUNK), jnp.int32),
        pltpu.VMEM((CHUNK, H), jnp.float32),
        pltpu.VMEM((CHUNK, H), jnp.float32),
        pltpu.VMEM_SHARED((N_ACC, H), jnp.float32),
        pltpu.SemaphoreType.DMA,
        pltpu.SemaphoreType.DMA,
        pltpu.SemaphoreType.DMA,
        pltpu.SemaphoreType.DMA,
    ],
)
def _sc_scatter(y_hbm, src_hbm, dst_hbm, zeros_hbm, out_hbm,
                src_v, dst_v, rows0_v, rows1_v, acc_sh,
                semg0, semg1, sems0, sems1):
    c = lax.axis_index("c")
    s = lax.axis_index("s")
    row0 = c * IDX_ROWS + s * EC_PT
    pltpu.sync_copy(src_hbm.at[pl.ds(row0, EC_PT)], src_v)
    pltpu.sync_copy(zeros_hbm.at[pl.ds(0, ROWS_PT)],
                    acc_sh.at[pl.ds(s * ROWS_PT, ROWS_PT)])
    plsc.subcore_barrier()

    HB = CHUNK // 2

    def _gather(t, buf, sem):
        pltpu.async_copy(y_hbm.at[src_v.at[t].at[pl.ds(0, HB)]], buf.at[pl.ds(0, HB)], sem)
        pltpu.async_copy(y_hbm.at[src_v.at[t].at[pl.ds(HB, HB)]], buf.at[pl.ds(HB, HB)], sem)

    def _gwait(t, buf, sem):
        pltpu.make_async_copy(y_hbm.at[src_v.at[t].at[pl.ds(0, HB)]], buf.at[pl.ds(0, HB)], sem).wait()
        pltpu.make_async_copy(y_hbm.at[src_v.at[t].at[pl.ds(HB, HB)]], buf.at[pl.ds(HB, HB)], sem).wait()

    _gather(0, rows0_v, semg0)

    @pl.loop(0, EC_PT, step=8)
    def _(j):
        @pl.when(j > 0)
        def _():
            pltpu.make_async_copy(rows1_v, acc_sh.at[dst_v.at[7]], sems1).wait()

        pltpu.sync_copy(dst_hbm.at[pl.ds(row0 + j, 8)], dst_v)
        for k in range(0, 8, 2):
            _gwait(j + k, rows0_v, semg0)
            pltpu.async_copy(rows0_v, acc_sh.at[dst_v.at[k]], sems0, add=True)
            if k > 0:
                pltpu.make_async_copy(rows1_v, acc_sh.at[dst_v.at[k - 1]], sems1).wait()
            _gather(j + k + 1, rows1_v, semg1)
            _gwait(j + k + 1, rows1_v, semg1)
            pltpu.async_copy(rows1_v, acc_sh.at[dst_v.at[k + 1]], sems1, add=True)
            pltpu.make_async_copy(rows0_v, acc_sh.at[dst_v.at[k]], sems0).wait()

            @pl.when(j + k + 2 < EC_PT)
            def _():
                _gather(j + k + 2, rows0_v, semg0)

    pltpu.make_async_copy(rows1_v, acc_sh.at[dst_v.at[7]], sems1).wait()
    plsc.subcore_barrier()
    pltpu.sync_copy(acc_sh.at[pl.ds(s * ROWS_PT, ROWS_PT)],
                    out_hbm.at[pl.ds(c * N_ACC + s * ROWS_PT, ROWS_PT)])


def _tc_pre(x_ref, w_ref, b_ref, g_ref, be_ref, o_ref):
    h = _mm(x_ref[...], w_ref[...]) + b_ref[...]
    o_ref[...] = jnp.maximum(_bn(h, g_ref[...], be_ref[...]), 0.0)


def _tc_a(h_ref, deg_ref, wsk_ref, bsk_ref, we0_ref, we1_ref, y_ref, hres_ref):
    h = h_ref[...]
    deg = deg_ref[...]
    hres_ref[...] = _mm(h, wsk_ref[...]) + bsk_ref[...]
    c0 = lax.rsqrt(jnp.clip(deg[0, 0:N], 1.0, None)).reshape(N, 1)
    c1 = lax.rsqrt(jnp.clip(deg[1, 0:N], 1.0, None)).reshape(N, 1)
    y_ref[0:N, :] = _mm(h * c0, we0_ref[...])
    y_ref[N:N_ACC, :] = jnp.zeros((N_ACC - N, H), jnp.float32)
    y_ref[N_ACC:N_ACC + N, :] = _mm(h * c1, we1_ref[...])
    y_ref[N_ACC + N:, :] = jnp.zeros((N_ACC - N, H), jnp.float32)


def _tc_b(s_ref, deg_ref, hres_ref, b0_ref, b1_ref, g_ref, be_ref, o_ref):
    deg = deg_ref[...]
    i0 = lax.rsqrt(jnp.clip(deg[0, N_ACC:N_ACC + N], 1.0, None)).reshape(N, 1)
    i1 = lax.rsqrt(jnp.clip(deg[1, N_ACC:N_ACC + N], 1.0, None)).reshape(N, 1)
    rel0 = s_ref[0:N, :] * i0 + b0_ref[...]
    rel1 = s_ref[N_ACC:N_ACC + N, :] * i1 + b1_ref[...]
    t = 0.5 * (rel0 + rel1) + hres_ref[...]
    t = _bn(t, g_ref[...], be_ref[...])
    o_ref[...] = jnp.where(t >= 0.0, t, NEG_SLOPE * t)


def _tc_head(h_ref, w1_ref, b1_ref, g_ref, be_ref, w2_ref, b2_ref, o_ref):
    t = _mm(h_ref[...], w1_ref[...]) + b1_ref[...]
    t = jnp.maximum(_bn(t, g_ref[...], be_ref[...]), 0.0)
    o_ref[...] = _mm(t, w2_ref[...]) + b2_ref[...]


def _call(body, out_shapes):
    return pl.pallas_call(body, out_shape=out_shapes)


def kernel(x, edge_index_e0, edge_index_e1, params):
    p = params
    f32 = jnp.float32
    pad = N + (jnp.arange(E_PAD - E, dtype=jnp.int32) % (N_ACC - N))
    s0p = jnp.concatenate([edge_index_e0[0], pad])
    d0p = jnp.concatenate([edge_index_e0[1], pad])
    s1p = jnp.concatenate([edge_index_e1[0], pad])
    d1p = jnp.concatenate([edge_index_e1[1], pad])

    deg_idx = jnp.stack([
        jnp.concatenate([s0p, d0p + N_ACC]),
        jnp.concatenate([s1p, d1p + N_ACC]),
    ]).reshape(4 * IDX_ROWS, CHUNK)
    src_all = jnp.concatenate([s0p, s1p + N_ACC]).reshape(2 * IDX_ROWS, CHUNK)
    dst_all = jnp.concatenate([d0p, d1p]).reshape(2 * IDX_ROWS, CHUNK)

    zeros1d = jnp.zeros((DEG_PT,), f32)
    ones1d = jnp.ones((CHUNK,), f32)
    zeros2d = jnp.zeros((ROWS_PT, H), f32)

    deg = _sc_degrees(deg_idx, zeros1d, ones1d)

    def r2(v):
        return v.reshape(1, -1)

    h = _call(_tc_pre, jax.ShapeDtypeStruct((N, H), f32))(
        x, p['W_fr'], r2(p['b_fr']), r2(p['g_fr']), r2(p['be_fr']))

    for lp in p['layers']:
        y, hres = _call(_tc_a, [
            jax.ShapeDtypeStruct((2 * N_ACC, H), f32),
            jax.ShapeDtypeStruct((N, H), f32),
        ])(h, deg, lp['W_skip'], r2(lp['b_skip']), lp['W_e0'], lp['W_e1'])
        s_out = _sc_scatter(y, src_all, dst_all, zeros2d)
        h = _call(_tc_b, jax.ShapeDtypeStruct((N, H), f32))(
            s_out, deg, hres, r2(lp['b_e0']), r2(lp['b_e1']),
            r2(lp['g']), r2(lp['be']))

    out = _call(_tc_head, jax.ShapeDtypeStruct((N, C), f32))(
        h, p['W_c1'], r2(p['b_c1']), r2(p['g_c']), r2(p['be_c']),
        p['W_c2'], r2(p['b_c2']))
    return out
